# baseline (speedup 1.0000x reference)
import functools

import jax
import jax.numpy as jnp
from jax import lax
from jax.experimental import pallas as pl
from jax.experimental.pallas import tpu as pltpu
from jax.experimental.pallas import tpu_sc as plsc

HIDDEN = 1024
FFN = 4096
NUM_EXPERTS = 8
TOP_K = 2
N = 4096
TB = 512
TM = 128
BF = 4096
NT = N // TM
NROWS = N * TOP_K
NTILES = NROWS // TM
W_ITEMS = NTILES + NUM_EXPERTS - 1

NW = 32
TOK_PER_W = N // NW
CHUNK = 32


def _router_body(x_ref, wg_ref, logits_ref, rank_ref, w0b_ref, w1b_ref,
                 sel_ref, counts_ref, carry_ref):
    i = pl.program_id(0)

    @pl.when(i == 0)
    def _init():
        carry_ref[...] = jnp.zeros_like(carry_ref)

    x = x_ref[...].astype(jnp.bfloat16)
    wg = wg_ref[...].astype(jnp.bfloat16)
    logits = jax.lax.dot_general(
        x, wg, (((1,), (1,)), ((), ())), preferred_element_type=jnp.float32
    )
    logits_ref[...] = logits
    lmax = jnp.max(logits, axis=1, keepdims=True)
    ex = jnp.exp(logits - lmax)
    p = ex / jnp.sum(ex, axis=1, keepdims=True)
    iota8 = jax.lax.broadcasted_iota(jnp.int32, (TB, NUM_EXPERTS), 1)
    m1 = jnp.max(p, axis=1, keepdims=True)
    i1 = jnp.min(jnp.where(p == m1, iota8, NUM_EXPERTS), axis=1, keepdims=True)
    pm = jnp.where(iota8 == i1, -1.0, p)
    m2 = jnp.max(pm, axis=1, keepdims=True)
    i2 = jnp.min(jnp.where(pm == m2, iota8, NUM_EXPERTS), axis=1, keepdims=True)
    s = m1 + m2
    oh1 = (iota8 == i1).astype(jnp.float32)
    oh2 = (iota8 == i2).astype(jnp.float32)
    w0b_ref[...] = jnp.broadcast_to(m1 / s, (TB, 16))
    w1b_ref[...] = jnp.broadcast_to(m2 / s, (TB, 16))
    sel_ref[...] = jnp.concatenate([i1, i2], axis=1)

    tri = (jax.lax.broadcasted_iota(jnp.int32, (TB, TB), 0)
           > jax.lax.broadcasted_iota(jnp.int32, (TB, TB), 1)).astype(jnp.bfloat16)
    both = (oh1 + oh2).astype(jnp.bfloat16)
    pre = jax.lax.dot_general(
        tri, both, (((1,), (0,)), ((), ())), preferred_element_type=jnp.float32
    )
    carry = carry_ref[...]
    r0 = jnp.sum((carry + pre) * oh1, axis=1, keepdims=True)
    r1 = jnp.sum((carry + pre + oh1) * oh2, axis=1, keepdims=True)
    rank_ref[...] = jnp.concatenate([r0, r1], axis=1)
    carry_ref[...] = carry + jnp.sum(oh1 + oh2, axis=0, keepdims=True)
    counts_ref[...] = carry_ref[...]


def _router(xf, Wg):
    return pl.pallas_call(
        _router_body,
        grid=(N // TB,),
        in_specs=[
            pl.BlockSpec((TB, HIDDEN), lambda i: (i, 0)),
            pl.BlockSpec((NUM_EXPERTS, HIDDEN), lambda i: (0, 0)),
        ],
        out_specs=[
            pl.BlockSpec((TB, NUM_EXPERTS), lambda i: (i, 0)),
            pl.BlockSpec((TB, TOP_K), lambda i: (i, 0)),
            pl.BlockSpec((TB, 16), lambda i: (i, 0)),
            pl.BlockSpec((TB, 16), lambda i: (i, 0)),
            pl.BlockSpec((TB, TOP_K), lambda i: (i, 0)),
            pl.BlockSpec((1, NUM_EXPERTS), lambda i: (0, 0)),
        ],
        out_shape=[
            jax.ShapeDtypeStruct((N, NUM_EXPERTS), jnp.float32),
            jax.ShapeDtypeStruct((N, TOP_K), jnp.float32),
            jax.ShapeDtypeStruct((N, 16), jnp.float32),
            jax.ShapeDtypeStruct((N, 16), jnp.float32),
            jax.ShapeDtypeStruct((N, TOP_K), jnp.int32),
            jax.ShapeDtypeStruct((1, NUM_EXPERTS), jnp.float32),
        ],
        scratch_shapes=[pltpu.VMEM((1, NUM_EXPERTS), jnp.float32)],
    )(xf, Wg)


def _finalize_body(counts_ref, sel_ref, rank_ref, pos0_ref, pos1_ref):
    counts = counts_ref[...]
    z = jnp.zeros_like(counts)
    acc = counts
    for sh in (1, 2, 4):
        shifted = jnp.concatenate(
            [z[:, :sh], acc[:, : NUM_EXPERTS - sh]], axis=1)
        acc = acc + shifted
    off = acc - counts
    iota8 = jax.lax.broadcasted_iota(jnp.int32, (TB, NUM_EXPERTS), 1)
    sel = sel_ref[...]
    rank = rank_ref[...]
    p0 = jnp.sum(jnp.where(iota8 == sel[:, 0:1], off, 0.0), axis=1, keepdims=True)
    p1 = jnp.sum(jnp.where(iota8 == sel[:, 1:2], off, 0.0), axis=1, keepdims=True)
    pos0_ref[...] = (p0 + rank[:, 0:1]).astype(jnp.int32)
    pos1_ref[...] = (p1 + rank[:, 1:2]).astype(jnp.int32)


def _finalize(counts, sel, rank):
    return pl.pallas_call(
        _finalize_body,
        grid=(N // TB,),
        in_specs=[
            pl.BlockSpec((1, NUM_EXPERTS), lambda i: (0, 0)),
            pl.BlockSpec((TB, TOP_K), lambda i: (i, 0)),
            pl.BlockSpec((TB, TOP_K), lambda i: (i, 0)),
        ],
        out_specs=[
            pl.BlockSpec((TB, 1), lambda i: (i, 0)),
            pl.BlockSpec((TB, 1), lambda i: (i, 0)),
        ],
        out_shape=[
            jax.ShapeDtypeStruct((N, 1), jnp.int32),
            jax.ShapeDtypeStruct((N, 1), jnp.int32),
        ],
    )(counts, sel, rank)


def _dispatch_sc(xf, pos0, pos1):
    mesh = plsc.VectorSubcoreMesh(core_axis_name="c", subcore_axis_name="s")

    @functools.partial(
        pl.kernel, mesh=mesh,
        out_type=jax.ShapeDtypeStruct((NROWS, HIDDEN), jnp.float32),
        scratch_types=[
            pltpu.VMEM((CHUNK, HIDDEN), jnp.float32),
            pltpu.VMEM((CHUNK,), jnp.int32),
            pltpu.VMEM((CHUNK,), jnp.int32),
            pltpu.SemaphoreType.DMA,
            pltpu.SemaphoreType.DMA,
        ],
    )
    def k(x_hbm, p0_hbm, p1_hbm, xs_hbm, xv, i0v, i1v, sem0, sem1):
        wid = lax.axis_index("s") * 2 + lax.axis_index("c")
        for c in range(TOK_PER_W // CHUNK):
            base = wid * TOK_PER_W + c * CHUNK
            pltpu.sync_copy(x_hbm.at[pl.ds(base, CHUNK)], xv)
            pltpu.sync_copy(p0_hbm.at[pl.ds(base, CHUNK)], i0v)
            pltpu.sync_copy(p1_hbm.at[pl.ds(base, CHUNK)], i1v)
            cp0 = pltpu.make_async_copy(xv, xs_hbm.at[i0v], sem0)
            cp1 = pltpu.make_async_copy(xv, xs_hbm.at[i1v], sem1)
            cp0.start()
            cp1.start()
            cp0.wait()
            cp1.wait()

    return k(xf, pos0, pos1)


def _gmm_body(g_ref, t_ref, lo_ref, hi_ref, xs_ref, w1_ref, w3_ref, w2_ref,
              out_ref):
    i = pl.program_id(0)
    f = pl.program_id(1)
    tile = t_ref[i]
    prev = t_ref[jnp.maximum(i - 1, 0)]
    first = jnp.logical_or(i == 0, tile != prev)

    @pl.when(jnp.logical_and(first, f == 0))
    def _init():
        out_ref[...] = jnp.zeros_like(out_ref)

    x = xs_ref[...].astype(jnp.bfloat16)
    r = tile * TM + jax.lax.broadcasted_iota(jnp.int32, (TM, 1), 0)
    valid = jnp.logical_and(r >= lo_ref[i], r < hi_ref[i])
    CH = 1024
    y = jnp.zeros((TM, HIDDEN), jnp.float32)
    for c in range(BF // CH):
        w1c = w1_ref[0, c * CH:(c + 1) * CH, :]
        w3c = w3_ref[0, c * CH:(c + 1) * CH, :]
        w2c = w2_ref[0, :, c * CH:(c + 1) * CH]
        a = jax.lax.dot_general(
            x, w1c, (((1,), (1,)), ((), ())), preferred_element_type=jnp.float32
        )
        b = jax.lax.dot_general(
            x, w3c, (((1,), (1,)), ((), ())), preferred_element_type=jnp.float32
        )
        hmid = jnp.where(valid, a * jax.nn.sigmoid(a) * b, 0.0).astype(jnp.bfloat16)
        y = y + jax.lax.dot_general(
            hmid, w2c, (((1,), (1,)), ((), ())), preferred_element_type=jnp.float32
        )
    out_ref[...] += y


def _gmm(xs, w1b, w3b, w2b, gids, tids, lo, hi):
    grid_spec = pltpu.PrefetchScalarGridSpec(
        num_scalar_prefetch=4,
        grid=(W_ITEMS, FFN // BF),
        in_specs=[
            pl.BlockSpec((TM, HIDDEN), lambda i, f, g, t, lo, hi: (t[i], 0)),
            pl.BlockSpec((1, BF, HIDDEN), lambda i, f, g, t, lo, hi: (g[i], f, 0)),
            pl.BlockSpec((1, BF, HIDDEN), lambda i, f, g, t, lo, hi: (g[i], f, 0)),
            pl.BlockSpec((1, HIDDEN, BF), lambda i, f, g, t, lo, hi: (g[i], 0, f)),
        ],
        out_specs=pl.BlockSpec((TM, HIDDEN), lambda i, f, g, t, lo, hi: (t[i], 0)),
    )
    return pl.pallas_call(
        _gmm_body,
        grid_spec=grid_spec,
        out_shape=jax.ShapeDtypeStruct((NROWS, HIDDEN), jnp.float32),
        compiler_params=pltpu.CompilerParams(
            dimension_semantics=("arbitrary", "arbitrary"),
            vmem_limit_bytes=100 * 1024 * 1024,
        ),
    )(gids, tids, lo, hi, xs, w1b, w3b, w2b)


def _combine_sc(ys, pos0, pos1, w0b, w1b):
    mesh = plsc.VectorSubcoreMesh(core_axis_name="c", subcore_axis_name="s")

    @functools.partial(
        pl.kernel, mesh=mesh,
        out_type=jax.ShapeDtypeStruct((N, HIDDEN), jnp.float32),
        scratch_types=[
            pltpu.VMEM((CHUNK, HIDDEN), jnp.float32),
            pltpu.VMEM((CHUNK, HIDDEN), jnp.float32),
            pltpu.VMEM((CHUNK, HIDDEN), jnp.float32),
            pltpu.VMEM((CHUNK,), jnp.int32),
            pltpu.VMEM((CHUNK,), jnp.int32),
            pltpu.VMEM((CHUNK, 16), jnp.float32),
            pltpu.VMEM((CHUNK, 16), jnp.float32),
            pltpu.SemaphoreType.DMA,
            pltpu.SemaphoreType.DMA,
        ],
    )
    def k(ys_hbm, p0_hbm, p1_hbm, w0_hbm, w1_hbm, out_hbm,
          y0v, y1v, ov, i0v, i1v, w0v, w1v, sem0, sem1):
        wid = lax.axis_index("s") * 2 + lax.axis_index("c")
        for c in range(TOK_PER_W // CHUNK):
            base = wid * TOK_PER_W + c * CHUNK
            pltpu.sync_copy(p0_hbm.at[pl.ds(base, CHUNK)], i0v)
            pltpu.sync_copy(p1_hbm.at[pl.ds(base, CHUNK)], i1v)
            pltpu.sync_copy(w0_hbm.at[pl.ds(base, CHUNK)], w0v)
            pltpu.sync_copy(w1_hbm.at[pl.ds(base, CHUNK)], w1v)
            cp0 = pltpu.make_async_copy(ys_hbm.at[i0v], y0v, sem0)
            cp1 = pltpu.make_async_copy(ys_hbm.at[i1v], y1v, sem1)
            cp0.start()
            cp1.start()
            cp0.wait()
            cp1.wait()

            def body(r, _):
                wa = w0v[r, :]
                wb = w1v[r, :]
                for j in range(HIDDEN // 16):
                    sl = pl.ds(j * 16, 16)
                    ov[r, sl] = wa * y0v[r, sl] + wb * y1v[r, sl]
                return _

            lax.fori_loop(0, CHUNK, body, 0)
            pltpu.sync_copy(ov, out_hbm.at[pl.ds(base, CHUNK)])

    return k(ys, pos0, pos1, w0b, w1b)


def _metadata(counts_f32):
    counts = jnp.round(counts_f32[0]).astype(jnp.int32)
    off = jnp.concatenate([jnp.zeros((1,), jnp.int32), jnp.cumsum(counts)])
    start_tile = off[:NUM_EXPERTS] // TM
    end_tile = jnp.maximum(off[1:] - 1, 0) // TM
    nt = jnp.where(counts > 0, end_tile - start_tile + 1, 0)
    base = jnp.concatenate([jnp.zeros((1,), jnp.int32), jnp.cumsum(nt)[:-1]])
    total = jnp.sum(nt)
    ar = jnp.arange(W_ITEMS, dtype=jnp.int32)
    gi = jnp.clip(
        jnp.sum((base[None, :] <= ar[:, None]).astype(jnp.int32), axis=1) - 1,
        0, NUM_EXPERTS - 1)
    vi = ar < total
    ti = start_tile[gi] + (ar - base[gi])
    tids = jnp.where(vi, ti, NTILES - 1)
    gids = jnp.where(vi, gi, NUM_EXPERTS - 1)
    lo = jnp.where(vi, off[gids], 0)
    hi = jnp.where(vi, off[gids + 1], 0)
    return gids, tids, lo, hi


def kernel(hidden_states, Wg, w1, w2, w3):
    b, s, h = hidden_states.shape
    xf = hidden_states.reshape(-1, h)
    logits, rank, w0b, w1b, sel, counts = _router(xf, Wg)
    pos0, pos1 = _finalize(counts, sel, rank)
    pos0 = pos0.reshape(-1)
    pos1 = pos1.reshape(-1)
    gids, tids, lo, hi = _metadata(counts)
    xs = _dispatch_sc(xf, pos0, pos1)
    ys = _gmm(
        xs,
        w1.astype(jnp.bfloat16),
        w3.astype(jnp.bfloat16),
        w2.astype(jnp.bfloat16),
        gids, tids, lo, hi,
    )
    final = _combine_sc(ys, pos0, pos1, w0b, w1b)
    return final.reshape(b, s, h), logits


# speedup vs baseline: 1.6189x; 1.6189x over previous
import functools

import jax
import jax.numpy as jnp
from jax import lax
from jax.experimental import pallas as pl
from jax.experimental.pallas import tpu as pltpu
from jax.experimental.pallas import tpu_sc as plsc

HIDDEN = 1024
FFN = 4096
NUM_EXPERTS = 8
TOP_K = 2
N = 4096
TB = 512
TM = 256
BF = 4096
NT = N // TM
NROWS = N * TOP_K
NTILES = NROWS // TM
W_ITEMS = NTILES + NUM_EXPERTS - 1

NW = 32
TOK_PER_W = N // NW
CHUNK = 32


def _router_body(x_ref, wg_ref, logits_ref, rank_ref, w0b_ref, w1b_ref,
                 sel_ref, counts_ref, carry_ref):
    i = pl.program_id(0)

    @pl.when(i == 0)
    def _init():
        carry_ref[...] = jnp.zeros_like(carry_ref)

    x = x_ref[...].astype(jnp.bfloat16)
    wg = wg_ref[...].astype(jnp.bfloat16)
    logits = jax.lax.dot_general(
        x, wg, (((1,), (1,)), ((), ())), preferred_element_type=jnp.float32
    )
    logits_ref[...] = logits
    lmax = jnp.max(logits, axis=1, keepdims=True)
    ex = jnp.exp(logits - lmax)
    p = ex / jnp.sum(ex, axis=1, keepdims=True)
    iota8 = jax.lax.broadcasted_iota(jnp.int32, (TB, NUM_EXPERTS), 1)
    m1 = jnp.max(p, axis=1, keepdims=True)
    i1 = jnp.min(jnp.where(p == m1, iota8, NUM_EXPERTS), axis=1, keepdims=True)
    pm = jnp.where(iota8 == i1, -1.0, p)
    m2 = jnp.max(pm, axis=1, keepdims=True)
    i2 = jnp.min(jnp.where(pm == m2, iota8, NUM_EXPERTS), axis=1, keepdims=True)
    s = m1 + m2
    oh1 = (iota8 == i1).astype(jnp.float32)
    oh2 = (iota8 == i2).astype(jnp.float32)
    w0b_ref[...] = jnp.broadcast_to(m1 / s, (TB, 16))
    w1b_ref[...] = jnp.broadcast_to(m2 / s, (TB, 16))
    sel_ref[...] = jnp.concatenate([i1, i2], axis=1)

    tri = (jax.lax.broadcasted_iota(jnp.int32, (TB, TB), 0)
           > jax.lax.broadcasted_iota(jnp.int32, (TB, TB), 1)).astype(jnp.bfloat16)
    both = (oh1 + oh2).astype(jnp.bfloat16)
    pre = jax.lax.dot_general(
        tri, both, (((1,), (0,)), ((), ())), preferred_element_type=jnp.float32
    )
    carry = carry_ref[...]
    r0 = jnp.sum((carry + pre) * oh1, axis=1, keepdims=True)
    r1 = jnp.sum((carry + pre + oh1) * oh2, axis=1, keepdims=True)
    rank_ref[...] = jnp.concatenate([r0, r1], axis=1)
    carry_ref[...] = carry + jnp.sum(oh1 + oh2, axis=0, keepdims=True)
    counts_ref[...] = carry_ref[...]


def _router(xf, Wg):
    return pl.pallas_call(
        _router_body,
        grid=(N // TB,),
        in_specs=[
            pl.BlockSpec((TB, HIDDEN), lambda i: (i, 0)),
            pl.BlockSpec((NUM_EXPERTS, HIDDEN), lambda i: (0, 0)),
        ],
        out_specs=[
            pl.BlockSpec((TB, NUM_EXPERTS), lambda i: (i, 0)),
            pl.BlockSpec((TB, TOP_K), lambda i: (i, 0)),
            pl.BlockSpec((TB, 16), lambda i: (i, 0)),
            pl.BlockSpec((TB, 16), lambda i: (i, 0)),
            pl.BlockSpec((TB, TOP_K), lambda i: (i, 0)),
            pl.BlockSpec((1, NUM_EXPERTS), lambda i: (0, 0)),
        ],
        out_shape=[
            jax.ShapeDtypeStruct((N, NUM_EXPERTS), jnp.float32),
            jax.ShapeDtypeStruct((N, TOP_K), jnp.float32),
            jax.ShapeDtypeStruct((N, 16), jnp.float32),
            jax.ShapeDtypeStruct((N, 16), jnp.float32),
            jax.ShapeDtypeStruct((N, TOP_K), jnp.int32),
            jax.ShapeDtypeStruct((1, NUM_EXPERTS), jnp.float32),
        ],
        scratch_shapes=[pltpu.VMEM((1, NUM_EXPERTS), jnp.float32)],
    )(xf, Wg)


def _finalize_body(counts_ref, sel_ref, rank_ref, pos0_ref, pos1_ref):
    counts = counts_ref[...]
    z = jnp.zeros_like(counts)
    acc = counts
    for sh in (1, 2, 4):
        shifted = jnp.concatenate(
            [z[:, :sh], acc[:, : NUM_EXPERTS - sh]], axis=1)
        acc = acc + shifted
    off = acc - counts
    iota8 = jax.lax.broadcasted_iota(jnp.int32, (TB, NUM_EXPERTS), 1)
    sel = sel_ref[...]
    rank = rank_ref[...]
    p0 = jnp.sum(jnp.where(iota8 == sel[:, 0:1], off, 0.0), axis=1, keepdims=True)
    p1 = jnp.sum(jnp.where(iota8 == sel[:, 1:2], off, 0.0), axis=1, keepdims=True)
    pos0_ref[...] = (p0 + rank[:, 0:1]).astype(jnp.int32)
    pos1_ref[...] = (p1 + rank[:, 1:2]).astype(jnp.int32)


def _finalize(counts, sel, rank):
    return pl.pallas_call(
        _finalize_body,
        grid=(N // TB,),
        in_specs=[
            pl.BlockSpec((1, NUM_EXPERTS), lambda i: (0, 0)),
            pl.BlockSpec((TB, TOP_K), lambda i: (i, 0)),
            pl.BlockSpec((TB, TOP_K), lambda i: (i, 0)),
        ],
        out_specs=[
            pl.BlockSpec((TB, 1), lambda i: (i, 0)),
            pl.BlockSpec((TB, 1), lambda i: (i, 0)),
        ],
        out_shape=[
            jax.ShapeDtypeStruct((N, 1), jnp.int32),
            jax.ShapeDtypeStruct((N, 1), jnp.int32),
        ],
    )(counts, sel, rank)


def _dispatch_sc(xf, pos0, pos1):
    mesh = plsc.VectorSubcoreMesh(core_axis_name="c", subcore_axis_name="s")

    @functools.partial(
        pl.kernel, mesh=mesh,
        out_type=jax.ShapeDtypeStruct((NROWS, HIDDEN), jnp.float32),
        scratch_types=[
            pltpu.VMEM((CHUNK, HIDDEN), jnp.float32),
            pltpu.VMEM((CHUNK,), jnp.int32),
            pltpu.VMEM((CHUNK,), jnp.int32),
            pltpu.SemaphoreType.DMA,
            pltpu.SemaphoreType.DMA,
        ],
    )
    def k(x_hbm, p0_hbm, p1_hbm, xs_hbm, xv, i0v, i1v, sem0, sem1):
        wid = lax.axis_index("s") * 2 + lax.axis_index("c")
        for c in range(TOK_PER_W // CHUNK):
            base = wid * TOK_PER_W + c * CHUNK
            pltpu.sync_copy(x_hbm.at[pl.ds(base, CHUNK)], xv)
            pltpu.sync_copy(p0_hbm.at[pl.ds(base, CHUNK)], i0v)
            pltpu.sync_copy(p1_hbm.at[pl.ds(base, CHUNK)], i1v)
            cp0 = pltpu.make_async_copy(xv, xs_hbm.at[i0v], sem0)
            cp1 = pltpu.make_async_copy(xv, xs_hbm.at[i1v], sem1)
            cp0.start()
            cp1.start()
            cp0.wait()
            cp1.wait()

    return k(xf, pos0, pos1)


def _gmm_body(g_ref, t_ref, lo_ref, hi_ref, xs_ref, w1_ref, w3_ref, w2_ref,
              out_ref):
    i = pl.program_id(0)
    f = pl.program_id(1)
    tile = t_ref[i]
    prev = t_ref[jnp.maximum(i - 1, 0)]
    first = jnp.logical_or(i == 0, tile != prev)

    @pl.when(jnp.logical_and(first, f == 0))
    def _init():
        out_ref[...] = jnp.zeros_like(out_ref)

    x = xs_ref[...].astype(jnp.bfloat16)
    r = tile * TM + jax.lax.broadcasted_iota(jnp.int32, (TM, 1), 0)
    valid = jnp.logical_and(r >= lo_ref[i], r < hi_ref[i])
    CH = 1024
    y = jnp.zeros((TM, HIDDEN), jnp.float32)
    for c in range(BF // CH):
        w1c = w1_ref[0, c * CH:(c + 1) * CH, :]
        w3c = w3_ref[0, c * CH:(c + 1) * CH, :]
        w2c = w2_ref[0, :, c * CH:(c + 1) * CH]
        a = jax.lax.dot_general(
            x, w1c, (((1,), (1,)), ((), ())), preferred_element_type=jnp.float32
        )
        b = jax.lax.dot_general(
            x, w3c, (((1,), (1,)), ((), ())), preferred_element_type=jnp.float32
        )
        hmid = jnp.where(valid, a * jax.nn.sigmoid(a) * b, 0.0).astype(jnp.bfloat16)
        y = y + jax.lax.dot_general(
            hmid, w2c, (((1,), (1,)), ((), ())), preferred_element_type=jnp.float32
        )
    out_ref[...] += y


def _gmm(xs, w1b, w3b, w2b, gids, tids, lo, hi):
    grid_spec = pltpu.PrefetchScalarGridSpec(
        num_scalar_prefetch=4,
        grid=(W_ITEMS, FFN // BF),
        in_specs=[
            pl.BlockSpec((TM, HIDDEN), lambda i, f, g, t, lo, hi: (t[i], 0)),
            pl.BlockSpec((1, BF, HIDDEN), lambda i, f, g, t, lo, hi: (g[i], f, 0)),
            pl.BlockSpec((1, BF, HIDDEN), lambda i, f, g, t, lo, hi: (g[i], f, 0)),
            pl.BlockSpec((1, HIDDEN, BF), lambda i, f, g, t, lo, hi: (g[i], 0, f)),
        ],
        out_specs=pl.BlockSpec((TM, HIDDEN), lambda i, f, g, t, lo, hi: (t[i], 0)),
    )
    return pl.pallas_call(
        _gmm_body,
        grid_spec=grid_spec,
        out_shape=jax.ShapeDtypeStruct((NROWS, HIDDEN), jnp.float32),
        compiler_params=pltpu.CompilerParams(
            dimension_semantics=("arbitrary", "arbitrary"),
            vmem_limit_bytes=100 * 1024 * 1024,
        ),
    )(gids, tids, lo, hi, xs, w1b, w3b, w2b)


def _combine_sc(ys, pos0, pos1, w0b, w1b):
    mesh = plsc.VectorSubcoreMesh(core_axis_name="c", subcore_axis_name="s")

    @functools.partial(
        pl.kernel, mesh=mesh,
        out_type=jax.ShapeDtypeStruct((N, HIDDEN), jnp.float32),
        scratch_types=[
            pltpu.VMEM((CHUNK, HIDDEN), jnp.float32),
            pltpu.VMEM((CHUNK, HIDDEN), jnp.float32),
            pltpu.VMEM((CHUNK, HIDDEN), jnp.float32),
            pltpu.VMEM((CHUNK,), jnp.int32),
            pltpu.VMEM((CHUNK,), jnp.int32),
            pltpu.VMEM((CHUNK, 16), jnp.float32),
            pltpu.VMEM((CHUNK, 16), jnp.float32),
            pltpu.SemaphoreType.DMA,
            pltpu.SemaphoreType.DMA,
        ],
    )
    def k(ys_hbm, p0_hbm, p1_hbm, w0_hbm, w1_hbm, out_hbm,
          y0v, y1v, ov, i0v, i1v, w0v, w1v, sem0, sem1):
        wid = lax.axis_index("s") * 2 + lax.axis_index("c")
        for c in range(TOK_PER_W // CHUNK):
            base = wid * TOK_PER_W + c * CHUNK
            pltpu.sync_copy(p0_hbm.at[pl.ds(base, CHUNK)], i0v)
            pltpu.sync_copy(p1_hbm.at[pl.ds(base, CHUNK)], i1v)
            pltpu.sync_copy(w0_hbm.at[pl.ds(base, CHUNK)], w0v)
            pltpu.sync_copy(w1_hbm.at[pl.ds(base, CHUNK)], w1v)
            cp0 = pltpu.make_async_copy(ys_hbm.at[i0v], y0v, sem0)
            cp1 = pltpu.make_async_copy(ys_hbm.at[i1v], y1v, sem1)
            cp0.start()
            cp1.start()
            cp0.wait()
            cp1.wait()

            def body(r, _):
                wa = w0v[r, :]
                wb = w1v[r, :]
                for j in range(HIDDEN // 16):
                    sl = pl.ds(j * 16, 16)
                    ov[r, sl] = wa * y0v[r, sl] + wb * y1v[r, sl]
                return _

            lax.fori_loop(0, CHUNK, body, 0)
            pltpu.sync_copy(ov, out_hbm.at[pl.ds(base, CHUNK)])

    return k(ys, pos0, pos1, w0b, w1b)


def _metadata(counts_f32):
    counts = jnp.round(counts_f32[0]).astype(jnp.int32)
    off = jnp.concatenate([jnp.zeros((1,), jnp.int32), jnp.cumsum(counts)])
    start_tile = off[:NUM_EXPERTS] // TM
    end_tile = jnp.maximum(off[1:] - 1, 0) // TM
    nt = jnp.where(counts > 0, end_tile - start_tile + 1, 0)
    base = jnp.concatenate([jnp.zeros((1,), jnp.int32), jnp.cumsum(nt)[:-1]])
    total = jnp.sum(nt)
    ar = jnp.arange(W_ITEMS, dtype=jnp.int32)
    gi = jnp.clip(
        jnp.sum((base[None, :] <= ar[:, None]).astype(jnp.int32), axis=1) - 1,
        0, NUM_EXPERTS - 1)
    vi = ar < total
    ti = start_tile[gi] + (ar - base[gi])
    tids = jnp.where(vi, ti, NTILES - 1)
    gids = jnp.where(vi, gi, NUM_EXPERTS - 1)
    lo = jnp.where(vi, off[gids], 0)
    hi = jnp.where(vi, off[gids + 1], 0)
    return gids, tids, lo, hi


def kernel(hidden_states, Wg, w1, w2, w3):
    b, s, h = hidden_states.shape
    xf = hidden_states.reshape(-1, h)
    logits, rank, w0b, w1b, sel, counts = _router(xf, Wg)
    pos0, pos1 = _finalize(counts, sel, rank)
    pos0 = pos0.reshape(-1)
    pos1 = pos1.reshape(-1)
    gids, tids, lo, hi = _metadata(counts)
    xs = _dispatch_sc(xf, pos0, pos1)
    ys = _gmm(
        xs,
        w1.astype(jnp.bfloat16),
        w3.astype(jnp.bfloat16),
        w2.astype(jnp.bfloat16),
        gids, tids, lo, hi,
    )
    final = _combine_sc(ys, pos0, pos1, w0b, w1b)
    return final.reshape(b, s, h), logits
